# baseline (speedup 1.0000x reference)
import functools

import jax
import jax.numpy as jnp
from jax import lax
from jax.experimental import pallas as pl
from jax.experimental.pallas import tpu as pltpu
from jax.experimental.pallas import tpu_sc as plsc

_E = 8
_T = 2048
_NW = 32
_L = 16
_ROWS_PER_W = 256
_CHUNK = 32


def _router_body(x_ref, w_ref, idx_ref, wgt_ref):
    x = x_ref[...]
    w = w_ref[...]
    logits = jax.lax.dot_general(
        w, x, (((1,), (1,)), ((), ())), preferred_element_type=jnp.float32)
    eidx = jax.lax.broadcasted_iota(jnp.int32, logits.shape, 0)
    m1 = jnp.max(logits, axis=0, keepdims=True)
    i1 = jnp.min(jnp.where(logits == m1, eidx, _E), axis=0, keepdims=True)
    masked = jnp.where(eidx == i1, -jnp.inf, logits)
    m2 = jnp.max(masked, axis=0, keepdims=True)
    i2 = jnp.min(jnp.where(masked == m2, eidx, _E), axis=0, keepdims=True)
    w1 = 1.0 / (1.0 + jnp.exp(m2 - m1))
    idx_ref[...] = jnp.concatenate([i1, i2], axis=0)
    wgt_ref[...] = jnp.concatenate([w1, 1.0 - w1], axis=0)


def _stream_body(hs_hbm, out_hbm, buf0, buf1, sem0, sem1):
    wid = lax.axis_index("s") * 2 + lax.axis_index("c")
    base = wid * _ROWS_PER_W
    nst = _ROWS_PER_W // _CHUNK
    cp0 = pltpu.async_copy(hs_hbm.at[pl.ds(base, _CHUNK)], buf0, sem0)

    def step(j, carry):
        nxt = base + (j + 1) * _CHUNK
        even = j % 2 == 0

        @pl.when(jnp.logical_and(even, j + 1 < nst))
        def _():
            pltpu.async_copy(hs_hbm.at[pl.ds(nxt, _CHUNK)], buf1, sem1)

        @pl.when(jnp.logical_and(jnp.logical_not(even), j + 1 < nst))
        def _():
            pltpu.async_copy(hs_hbm.at[pl.ds(nxt, _CHUNK)], buf0, sem0)

        @pl.when(even)
        def _():
            pltpu.make_async_copy(hs_hbm.at[pl.ds(base, _CHUNK)], buf0, sem0).wait()

        @pl.when(jnp.logical_not(even))
        def _():
            pltpu.make_async_copy(hs_hbm.at[pl.ds(base, _CHUNK)], buf1, sem1).wait()

        return carry

    lax.fori_loop(0, nst, step, 0)
    del cp0
    pltpu.sync_copy(buf0.at[0, pl.ds(0, _L)], out_hbm.at[pl.ds(wid * _L, _L)])


def kernel(hidden_states, weights):
    b, s, d = hidden_states.shape
    n = b * s
    hs = hidden_states.reshape(n, d)
    probe = functools.partial(
        pl.kernel,
        out_type=[jax.ShapeDtypeStruct((_NW * _L,), jnp.float32)],
        mesh=plsc.VectorSubcoreMesh(core_axis_name="c", subcore_axis_name="s"),
        scratch_types=[
            pltpu.VMEM((_CHUNK, 768), jnp.float32),
            pltpu.VMEM((_CHUNK, 768), jnp.float32),
            pltpu.SemaphoreType.DMA,
            pltpu.SemaphoreType.DMA,
        ],
    )(_stream_body)
    (dummy,) = probe(hs)
    idx_t, wgt_t = pl.pallas_call(
        _router_body,
        grid=(n // _T,),
        in_specs=[
            pl.BlockSpec((_T, d), lambda i: (i, 0)),
            pl.BlockSpec((_E, d), lambda i: (0, 0)),
        ],
        out_specs=[
            pl.BlockSpec((2, _T), lambda i: (0, i)),
            pl.BlockSpec((2, _T), lambda i: (0, i)),
        ],
        out_shape=[
            jax.ShapeDtypeStruct((2, n), jnp.int32),
            jax.ShapeDtypeStruct((2, n), jnp.float32),
        ],
    )(hs, weights)

    aux = jnp.minimum(jnp.abs(dummy[0]) * 1e-30, 0.0)
    return idx_t.T, wgt_t.T, aux


# speedup vs baseline: 1.7462x; 1.7462x over previous
import jax
import jax.numpy as jnp
from jax.experimental import pallas as pl

_E = 8
_T = 2048


def _top2(logits):
    eidx = jax.lax.broadcasted_iota(jnp.int32, logits.shape, 0)
    m1 = jnp.max(logits, axis=0, keepdims=True)
    i1 = jnp.min(jnp.where(logits == m1, eidx, _E), axis=0, keepdims=True)
    masked = jnp.where(eidx == i1, -jnp.inf, logits)
    m2 = jnp.max(masked, axis=0, keepdims=True)
    i2 = jnp.min(jnp.where(masked == m2, eidx, _E), axis=0, keepdims=True)
    w1 = 1.0 / (1.0 + jnp.exp(m2 - m1))
    return (jnp.concatenate([i1, i2], axis=0),
            jnp.concatenate([w1, 1.0 - w1], axis=0))


def _router_body(x0_ref, x1_ref, w_ref, idx_ref, wgt_ref):
    w = w_ref[...]
    dn = (((1,), (1,)), ((), ()))
    lg0 = jax.lax.dot_general(w, x0_ref[...], dn, preferred_element_type=jnp.float32)
    i0, g0 = _top2(lg0)
    idx_ref[:, 0:_T] = i0
    wgt_ref[:, 0:_T] = g0
    lg1 = jax.lax.dot_general(w, x1_ref[...], dn, preferred_element_type=jnp.float32)
    i1, g1 = _top2(lg1)
    idx_ref[:, _T:2 * _T] = i1
    wgt_ref[:, _T:2 * _T] = g1


def kernel(hidden_states, weights):
    b, s, d = hidden_states.shape
    n = b * s
    hs = hidden_states.reshape(n, d)
    nblk = n // _T
    idx_t, wgt_t = pl.pallas_call(
        _router_body,
        grid=(nblk // 2,),
        in_specs=[
            pl.BlockSpec((_T, d), lambda i: (2 * i, 0)),
            pl.BlockSpec((_T, d), lambda i: (2 * i + 1, 0)),
            pl.BlockSpec((_E, d), lambda i: (0, 0)),
        ],
        out_specs=[
            pl.BlockSpec((2, 2 * _T), lambda i: (0, i)),
            pl.BlockSpec((2, 2 * _T), lambda i: (0, i)),
        ],
        out_shape=[
            jax.ShapeDtypeStruct((2, n), jnp.int32),
            jax.ShapeDtypeStruct((2, n), jnp.float32),
        ],
    )(hs, hs, weights)
    return idx_t.T, wgt_t.T, jnp.float32(0.0)
